# baseline (speedup 1.0000x reference)
import functools

import jax
import jax.numpy as jnp
from jax import lax
from jax.experimental import pallas as pl
from jax.experimental.pallas import tpu as pltpu
from jax.experimental.pallas import tpu_sc as plsc

_T, _D, _E, _H = 2048, 768, 64, 1024
_C = 64
_S = _E * _C
_NC, _NS = 2, 16
_NW = _NC * _NS


def _router_body(x_ref, rw_ref, slot_ref, gain_ref):
    logits = jnp.dot(x_ref[...], rw_ref[...], preferred_element_type=jnp.float32)
    m = jnp.max(logits, axis=-1, keepdims=True)
    ex = jnp.exp(logits - m)
    s = jnp.sum(ex, axis=-1, keepdims=True)
    probs = ex / s
    pm = jnp.max(probs, axis=-1, keepdims=True)
    iota_e = lax.broadcasted_iota(jnp.int32, (_T, _E), 1)
    e = jnp.min(jnp.where(probs == pm, iota_e, _E), axis=-1, keepdims=True)
    onehot = (iota_e == e).astype(jnp.float32)

    chunk = 256
    r = lax.broadcasted_iota(jnp.int32, (chunk, chunk), 0)
    c = lax.broadcasted_iota(jnp.int32, (chunk, chunk), 1)
    tri = (r > c).astype(jnp.float32)
    counts = jnp.zeros((1, _E), jnp.float32)
    pos_cols = []
    for i in range(_T // chunk):
        oh = onehot[i * chunk:(i + 1) * chunk, :]
        pos_chunk = jnp.dot(tri, oh, preferred_element_type=jnp.float32) + counts
        pos_cols.append(jnp.sum(pos_chunk * oh, axis=-1, keepdims=True))
        counts = counts + jnp.sum(oh, axis=0, keepdims=True)
    pos = jnp.concatenate(pos_cols, axis=0).astype(jnp.int32)

    keep = pos < _C
    slot_ref[...] = jnp.where(keep, e * _C + pos, -1)
    gain_ref[...] = jnp.where(keep, pm, 0.0)


def _router(x, router_w):
    return pl.pallas_call(
        _router_body,
        out_shape=(
            jax.ShapeDtypeStruct((_T, 1), jnp.int32),
            jax.ShapeDtypeStruct((_T, 1), jnp.float32),
        ),
    )(x, router_w)


def _dispatch_body(slot_hbm, x_hbm, ein_hbm, slots_v, tok_v, rows_v, sem):
    wid = lax.axis_index("s") * _NC + lax.axis_index("c")
    lo = wid * (_S // _NW)
    pltpu.sync_copy(slot_hbm, slots_v)
    zero16 = jnp.zeros((16,), jnp.int32)
    for i in range(_S // _NW // 16):
        tok_v[pl.ds(i * 16, 16)] = zero16

    def scan(i, carry):
        sl = slots_v[pl.ds(i * 16, 16)]
        tok = lax.broadcasted_iota(jnp.int32, (16,), 0) + i * 16
        m = (sl >= lo) & (sl < lo + _S // _NW)
        plsc.store_scatter(tok_v, [sl - lo], tok, mask=m)
        return carry

    lax.fori_loop(0, _T // 16, scan, 0)
    pltpu.async_copy(x_hbm.at[tok_v], rows_v, sem).wait()
    pltpu.sync_copy(rows_v, ein_hbm.at[pl.ds(lo, _S // _NW)])


def _dispatch(slot, x):
    mesh = plsc.VectorSubcoreMesh(core_axis_name="c", subcore_axis_name="s",
                                   num_cores=_NC, num_subcores=_NS)
    return pl.kernel(
        _dispatch_body,
        mesh=mesh,
        compiler_params=pltpu.CompilerParams(needs_layout_passes=False),
        out_type=jax.ShapeDtypeStruct((_S, _D), jnp.float32),
        scratch_types=[
            pltpu.VMEM((_T,), jnp.int32),
            pltpu.VMEM((_S // _NW,), jnp.int32),
            pltpu.VMEM((_S // _NW, _D), jnp.float32),
            pltpu.SemaphoreType.DMA,
        ],
    )(slot, x)


def _mlp_body(ein_ref, w1_ref, w2_ref, out_ref):
    h = jnp.maximum(
        jnp.dot(ein_ref[...], w1_ref[0], preferred_element_type=jnp.float32), 0.0)
    out_ref[...] = jnp.dot(h, w2_ref[0], preferred_element_type=jnp.float32)


def _mlp(ein, w1, w2):
    return pl.pallas_call(
        _mlp_body,
        grid=(_E,),
        in_specs=[
            pl.BlockSpec((_C, _D), lambda e: (e, 0)),
            pl.BlockSpec((1, _D, _H), lambda e: (e, 0, 0)),
            pl.BlockSpec((1, _H, _D), lambda e: (e, 0, 0)),
        ],
        out_specs=pl.BlockSpec((_C, _D), lambda e: (e, 0)),
        out_shape=jax.ShapeDtypeStruct((_S, _D), jnp.float32),
    )(ein, w1, w2)


def _combine_body(slot_hbm, gain_hbm, eout_hbm, out_hbm, slot_v, gain_v, rows_v, sem):
    wid = lax.axis_index("s") * _NC + lax.axis_index("c")
    tpw = _T // _NW
    base = wid * tpw
    pltpu.sync_copy(slot_hbm.at[pl.ds(base, tpw)], slot_v)
    pltpu.sync_copy(gain_hbm.at[pl.ds(base, tpw)], gain_v)
    for i in range(tpw // 16):
        s = slot_v[pl.ds(i * 16, 16)]
        slot_v[pl.ds(i * 16, 16)] = jnp.maximum(s, 0)
    pltpu.async_copy(eout_hbm.at[slot_v], rows_v, sem).wait()

    def scale(j, carry):
        g = plsc.load_gather(gain_v, [jnp.full((16,), j, jnp.int32)])
        for cc in range(_D // 16):
            rows_v[j, pl.ds(cc * 16, 16)] = rows_v[j, pl.ds(cc * 16, 16)] * g
        return carry

    lax.fori_loop(0, tpw, scale, 0)
    pltpu.sync_copy(rows_v, out_hbm.at[pl.ds(base, tpw)])


def _combine(slot, gain, eout):
    mesh = plsc.VectorSubcoreMesh(core_axis_name="c", subcore_axis_name="s",
                                   num_cores=_NC, num_subcores=_NS)
    return pl.kernel(
        _combine_body,
        mesh=mesh,
        compiler_params=pltpu.CompilerParams(needs_layout_passes=False),
        out_type=jax.ShapeDtypeStruct((_T, _D), jnp.float32),
        scratch_types=[
            pltpu.VMEM((_T // _NW,), jnp.int32),
            pltpu.VMEM((_T // _NW,), jnp.float32),
            pltpu.VMEM((_T // _NW, _D), jnp.float32),
            pltpu.SemaphoreType.DMA,
        ],
    )(slot, gain, eout)


def kernel(x, router_w, w1, w2):
    slot2d, gain2d = _router(x, router_w)
    slot = slot2d.reshape(_T)
    gain = gain2d.reshape(_T)
    ein = _dispatch(slot, x)
    eout = _mlp(ein, w1, w2)
    return _combine(slot, gain, eout)


# speedup vs baseline: 1.5053x; 1.5053x over previous
import functools

import jax
import jax.numpy as jnp
from jax import lax
from jax.experimental import pallas as pl
from jax.experimental.pallas import tpu as pltpu
from jax.experimental.pallas import tpu_sc as plsc

_T, _D, _E, _H = 2048, 768, 64, 1024
_C = 64
_S = _E * _C
_NC, _NS = 2, 16
_NW = _NC * _NS


def _router_body(x_ref, rw_ref, slot_ref, gain_ref):
    logits = jnp.dot(x_ref[...], rw_ref[...], preferred_element_type=jnp.float32)
    m = jnp.max(logits, axis=-1, keepdims=True)
    ex = jnp.exp(logits - m)
    s = jnp.sum(ex, axis=-1, keepdims=True)
    probs = ex / s
    pm = jnp.max(probs, axis=-1, keepdims=True)
    iota_e = lax.broadcasted_iota(jnp.int32, (_T, _E), 1)
    e = jnp.min(jnp.where(probs == pm, iota_e, _E), axis=-1, keepdims=True)
    onehot = (iota_e == e).astype(jnp.float32)

    chunk = 256
    r = lax.broadcasted_iota(jnp.int32, (chunk, chunk), 0)
    c = lax.broadcasted_iota(jnp.int32, (chunk, chunk), 1)
    tri = (r > c).astype(jnp.float32)
    counts = jnp.zeros((1, _E), jnp.float32)
    pos_cols = []
    for i in range(_T // chunk):
        oh = onehot[i * chunk:(i + 1) * chunk, :]
        pos_chunk = jnp.dot(tri, oh, preferred_element_type=jnp.float32) + counts
        pos_cols.append(jnp.sum(pos_chunk * oh, axis=-1, keepdims=True))
        counts = counts + jnp.sum(oh, axis=0, keepdims=True)
    pos = jnp.concatenate(pos_cols, axis=0).astype(jnp.int32)

    keep = pos < _C
    slot_ref[...] = jnp.where(keep, e * _C + pos, -1)
    gain_ref[...] = jnp.where(keep, pm, 0.0)


def _router(x, router_w):
    return pl.pallas_call(
        _router_body,
        out_shape=(
            jax.ShapeDtypeStruct((_T, 1), jnp.int32),
            jax.ShapeDtypeStruct((_T, 1), jnp.float32),
        ),
    )(x, router_w)


def _dispatch_body(slot_hbm, x_hbm, ein_hbm, slots_v, tok_v, rows_v, sem):
    wid = lax.axis_index("s") * _NC + lax.axis_index("c")
    lo = wid * (_S // _NW)
    pltpu.sync_copy(slot_hbm, slots_v)
    base = lo & (_T - 1)
    for i in range(_S // _NW // 16):
        tok_v[pl.ds(i * 16, 16)] = lax.broadcasted_iota(jnp.int32, (16,), 0) + (base + i * 16)

    def scan(i, carry):
        sl = slots_v[pl.ds(i * 16, 16)]
        tok = lax.broadcasted_iota(jnp.int32, (16,), 0) + i * 16
        m = (sl >= lo) & (sl < lo + _S // _NW)
        plsc.store_scatter(tok_v, [sl - lo], tok, mask=m)
        return carry

    lax.fori_loop(0, _T // 16, scan, 0)
    pltpu.async_copy(x_hbm.at[tok_v], rows_v, sem).wait()
    pltpu.sync_copy(rows_v, ein_hbm.at[pl.ds(lo, _S // _NW)])


def _dispatch(slot, x):
    mesh = plsc.VectorSubcoreMesh(core_axis_name="c", subcore_axis_name="s",
                                   num_cores=_NC, num_subcores=_NS)
    return pl.kernel(
        _dispatch_body,
        mesh=mesh,
        compiler_params=pltpu.CompilerParams(needs_layout_passes=False),
        out_type=jax.ShapeDtypeStruct((_S, _D), jnp.float32),
        scratch_types=[
            pltpu.VMEM((_T,), jnp.int32),
            pltpu.VMEM((_S // _NW,), jnp.int32),
            pltpu.VMEM((_S // _NW, _D), jnp.float32),
            pltpu.SemaphoreType.DMA,
        ],
    )(slot, x)


def _mlp_body(ein_ref, w1_ref, w2_ref, out_ref):
    h = jnp.maximum(
        jnp.dot(ein_ref[...], w1_ref[0], preferred_element_type=jnp.float32), 0.0)
    out_ref[...] = jnp.dot(h, w2_ref[0], preferred_element_type=jnp.float32)


def _mlp(ein, w1, w2):
    return pl.pallas_call(
        _mlp_body,
        grid=(_E,),
        in_specs=[
            pl.BlockSpec((_C, _D), lambda e: (e, 0)),
            pl.BlockSpec((1, _D, _H), lambda e: (e, 0, 0)),
            pl.BlockSpec((1, _H, _D), lambda e: (e, 0, 0)),
        ],
        out_specs=pl.BlockSpec((_C, _D), lambda e: (e, 0)),
        out_shape=jax.ShapeDtypeStruct((_S, _D), jnp.float32),
    )(ein, w1, w2)


def _combine_body(slot_hbm, gain_hbm, eout_hbm, out_hbm, slot_v, gain_v, rows_v, sem):
    wid = lax.axis_index("s") * _NC + lax.axis_index("c")
    tpw = _T // _NW
    base = wid * tpw
    pltpu.sync_copy(slot_hbm.at[pl.ds(base, tpw)], slot_v)
    pltpu.sync_copy(gain_hbm.at[pl.ds(base, tpw)], gain_v)
    for i in range(tpw // 16):
        s = slot_v[pl.ds(i * 16, 16)]
        slot_v[pl.ds(i * 16, 16)] = jnp.maximum(s, 0)
    pltpu.async_copy(eout_hbm.at[slot_v], rows_v, sem).wait()

    def scale(j, carry):
        g = plsc.load_gather(gain_v, [jnp.full((16,), j, jnp.int32)])
        for cc in range(_D // 16):
            rows_v[j, pl.ds(cc * 16, 16)] = rows_v[j, pl.ds(cc * 16, 16)] * g
        return carry

    lax.fori_loop(0, tpw, scale, 0)
    pltpu.sync_copy(rows_v, out_hbm.at[pl.ds(base, tpw)])


def _combine(slot, gain, eout):
    mesh = plsc.VectorSubcoreMesh(core_axis_name="c", subcore_axis_name="s",
                                   num_cores=_NC, num_subcores=_NS)
    return pl.kernel(
        _combine_body,
        mesh=mesh,
        compiler_params=pltpu.CompilerParams(needs_layout_passes=False),
        out_type=jax.ShapeDtypeStruct((_T, _D), jnp.float32),
        scratch_types=[
            pltpu.VMEM((_T // _NW,), jnp.int32),
            pltpu.VMEM((_T // _NW,), jnp.float32),
            pltpu.VMEM((_T // _NW, _D), jnp.float32),
            pltpu.SemaphoreType.DMA,
        ],
    )(slot, gain, eout)


def kernel(x, router_w, w1, w2):
    slot2d, gain2d = _router(x, router_w)
    slot = slot2d.reshape(_T)
    gain = gain2d.reshape(_T)
    ein = _dispatch(slot, x)
    eout = _mlp(ein, w1, w2)
    return _combine(slot, gain, eout)
